# Initial kernel scaffold; baseline (speedup 1.0000x reference)
import jax
import jax.numpy as jnp
from jax.experimental import pallas as pl


def kernel(q, k, v, idx_map):
    raise NotImplementedError("write your pallas kernel here")


# speedup vs baseline: 2.5061x; 2.5061x over previous
import functools
import math

import jax
import jax.numpy as jnp
from jax.experimental import pallas as pl
from jax.experimental.pallas import tpu as pltpu


def _arr2_offsets(S, levels):
    ioff2 = {2: 0}
    for l in range(3, levels):
        ioff2[l] = ioff2[l - 1] + (S >> (l - 1))
    return ioff2


def _parent_level(kc, vc, scale):
    kp0 = 0.5 * jnp.sum(kc, axis=1)
    vp0 = 0.5 * jnp.sum(vc, axis=1)
    s_self = jnp.sum(kp0 * kp0, axis=-1, keepdims=True) * scale
    s_pair = jnp.sum(kp0[:, None] * kc, axis=-1, keepdims=True) * scale
    m = jnp.maximum(s_self, jnp.max(s_pair, axis=1))
    e_self = jnp.exp(s_self - m)
    e_pair = jnp.exp(s_pair - m[:, None])
    denom = e_self + jnp.sum(e_pair, axis=1) + 1e-9
    kp = (e_self * kp0 + jnp.sum(e_pair * kc, axis=1)) / denom
    vp = (e_self * vp0 + jnp.sum(e_pair * vc, axis=1)) / denom
    return kp, vp


def _level1_body(kref, vref, k1ref, v1ref, *, Ts, Hc, D, scale):
    kc = kref[0].reshape(Ts // 2, 2, Hc, D)
    vc = vref[0].reshape(Ts // 2, 2, Hc, D)
    kp, vp = _parent_level(kc, vc, scale)
    k1ref[0] = kp
    v1ref[0] = vp


def _upper_body(k1ref, v1ref, k2ref, v2ref, *, S, Hc, D, levels, scale):
    ioff2 = _arr2_offsets(S, levels)
    for l in range(2, levels):
        n = S >> l
        if l == 2:
            kc = k1ref[0]
            vc = v1ref[0]
        else:
            kc = k2ref[0, ioff2[l - 1]:ioff2[l - 1] + 2 * n]
            vc = v2ref[0, ioff2[l - 1]:ioff2[l - 1] + 2 * n]
        kp, vp = _parent_level(kc.reshape(n, 2, Hc, D),
                               vc.reshape(n, 2, Hc, D), scale)
        k2ref[0, ioff2[l]:ioff2[l] + n] = kp
        v2ref[0, ioff2[l]:ioff2[l] + n] = vp


def _attn_body(*refs, T, H, D, levels, lt, scale):
    cols = levels + 1
    nlv = levels - 1
    qref, kref, vref, bref = refs[:4]
    knrefs = refs[4:4 + nlv]
    vnrefs = refs[4 + nlv:4 + 2 * nlv]
    oref = refs[4 + 2 * nlv]

    q = qref[0]
    kleaf = kref[0]
    vleaf = vref[0]
    bias = bref[...]

    def pair_swap(x):
        n = x.shape[0]
        r = x.reshape(n // 2, 2, H, D)
        return jnp.concatenate([r[:, 1:2], r[:, 0:1]], axis=1).reshape(n, H, D)

    m_run = None
    d_run = None
    acc = None
    for c in range(cols):
        bias_c = bias[:, c:c + 1].reshape(T, 1, 1)
        if c == 0:
            s = jnp.sum(q * kleaf, axis=-1, keepdims=True) * scale + bias_c
            m_run = s
            d_run = jnp.ones_like(s)
            acc = vleaf
            continue
        l = c - 1
        if l < lt:
            kl = kleaf if l == 0 else knrefs[l - 1][0]
            vl = vleaf if l == 0 else vnrefs[l - 1][0]
            ks = pair_swap(kl)
            vs = pair_swap(vl)
            g = 1 << l
            qg = q.reshape(T >> l, g, H, D)
            s = (jnp.sum(qg * ks[:, None], axis=-1, keepdims=True)
                 * scale).reshape(T, H, 1) + bias_c
            m_new = jnp.maximum(m_run, s)
            alpha = jnp.exp(m_run - m_new)
            e = jnp.exp(s - m_new)
            eg = e.reshape(T >> l, g, H, 1)
            contrib = (eg * vs[:, None]).reshape(T, H, D)
        else:
            ks = knrefs[l - 1][0]
            vs = vnrefs[l - 1][0]
            s = jnp.sum(q * ks, axis=-1, keepdims=True) * scale + bias_c
            m_new = jnp.maximum(m_run, s)
            alpha = jnp.exp(m_run - m_new)
            e = jnp.exp(s - m_new)
            contrib = e * vs
        d_run = d_run * alpha + e
        acc = acc * alpha + contrib
        m_run = m_new
    oref[0] = acc / d_run


def kernel(q, k, v, idx_map):
    B, S, H, D = q.shape
    cols = idx_map.shape[1]
    levels = cols - 1
    scale = 1.0 / math.sqrt(D)
    f32 = jnp.float32
    Hc = 8 if H % 8 == 0 else H
    ioff2 = _arr2_offsets(S, levels)
    par = pltpu.CompilerParams(dimension_semantics=("parallel",) * 3)
    par2 = pltpu.CompilerParams(dimension_semantics=("parallel", "parallel"))

    k = k.astype(f32)
    v = v.astype(f32)
    bias = jnp.where(idx_map < 0, -jnp.inf, 0.0).astype(f32)

    Ts = min(512, S)
    lvl1 = pl.pallas_call(
        functools.partial(_level1_body, Ts=Ts, Hc=Hc, D=D, scale=scale),
        grid=(B, S // Ts, H // Hc),
        in_specs=[
            pl.BlockSpec((1, Ts, Hc, D), lambda b, s, h: (b, s, h, 0)),
            pl.BlockSpec((1, Ts, Hc, D), lambda b, s, h: (b, s, h, 0)),
        ],
        out_specs=[
            pl.BlockSpec((1, Ts // 2, Hc, D), lambda b, s, h: (b, s, h, 0)),
            pl.BlockSpec((1, Ts // 2, Hc, D), lambda b, s, h: (b, s, h, 0)),
        ],
        out_shape=[
            jax.ShapeDtypeStruct((B, S // 2, H, D), f32),
            jax.ShapeDtypeStruct((B, S // 2, H, D), f32),
        ],
        compiler_params=par,
    )
    karr1, varr1 = lvl1(k, v)

    upper = pl.pallas_call(
        functools.partial(_upper_body, S=S, Hc=Hc, D=D, levels=levels,
                          scale=scale),
        grid=(B, H // Hc),
        in_specs=[
            pl.BlockSpec((1, S // 2, Hc, D), lambda b, h: (b, 0, h, 0)),
            pl.BlockSpec((1, S // 2, Hc, D), lambda b, h: (b, 0, h, 0)),
        ],
        out_specs=[
            pl.BlockSpec((1, S // 2, Hc, D), lambda b, h: (b, 0, h, 0)),
            pl.BlockSpec((1, S // 2, Hc, D), lambda b, h: (b, 0, h, 0)),
        ],
        out_shape=[
            jax.ShapeDtypeStruct((B, S // 2, H, D), f32),
            jax.ShapeDtypeStruct((B, S // 2, H, D), f32),
        ],
        compiler_params=par2,
    )
    karr2, varr2 = upper(karr1, varr1)

    T = min(128, S)
    lt = int(math.log2(T))
    nblocks = S // T

    node_specs = []
    for l in range(1, levels):
        if l == 1:
            node_specs.append(pl.BlockSpec(
                (1, T >> 1, H, D), lambda b, t: (b, t, 0, 0)))
        elif l < lt:
            blk = T >> l
            base = ioff2[l] // blk
            node_specs.append(pl.BlockSpec(
                (1, blk, H, D),
                lambda b, t, base=base: (b, base + t, 0, 0)))
        else:
            node_specs.append(pl.BlockSpec(
                (1, 1, H, D),
                lambda b, t, l=l, off=ioff2[l]: (b, off + ((t >> (l - lt)) ^ 1),
                                                 0, 0)))

    attn = pl.pallas_call(
        functools.partial(_attn_body, T=T, H=H, D=D, levels=levels, lt=lt,
                          scale=scale),
        grid=(B, nblocks),
        in_specs=[
            pl.BlockSpec((1, T, H, D), lambda b, t: (b, t, 0, 0)),
            pl.BlockSpec((1, T, H, D), lambda b, t: (b, t, 0, 0)),
            pl.BlockSpec((1, T, H, D), lambda b, t: (b, t, 0, 0)),
            pl.BlockSpec((T, cols), lambda b, t: (t, 0)),
        ] + node_specs + node_specs,
        out_specs=pl.BlockSpec((1, T, H, D), lambda b, t: (b, t, 0, 0)),
        out_shape=jax.ShapeDtypeStruct((B, S, H, D), f32),
        compiler_params=par2,
    )
    nreps = levels - 1
    karrs = [karr1 if l == 1 else karr2 for l in range(1, levels)]
    varrs = [varr1 if l == 1 else varr2 for l in range(1, levels)]
    out = attn(q.astype(f32), k, v, bias, *karrs, *varrs)
    return out.astype(q.dtype)


# speedup vs baseline: 5.1355x; 2.0493x over previous
import functools
import math

import jax
import jax.numpy as jnp
from jax import lax
from jax.experimental import pallas as pl
from jax.experimental.pallas import tpu as pltpu


def _arr2_offsets(S, levels):
    ioff2 = {2: 0}
    for l in range(3, levels):
        ioff2[l] = ioff2[l - 1] + (S >> (l - 1))
    return ioff2


def _parent_level(kc, vc, scale):
    kp0 = 0.5 * jnp.sum(kc, axis=1)
    vp0 = 0.5 * jnp.sum(vc, axis=1)
    s_self = jnp.sum(kp0 * kp0, axis=-1, keepdims=True) * scale
    s_pair = jnp.sum(kp0[:, None] * kc, axis=-1, keepdims=True) * scale
    m = jnp.maximum(s_self, jnp.max(s_pair, axis=1))
    e_self = jnp.exp(s_self - m)
    e_pair = jnp.exp(s_pair - m[:, None])
    denom = e_self + jnp.sum(e_pair, axis=1) + 1e-9
    kp = (e_self * kp0 + jnp.sum(e_pair * kc, axis=1)) / denom
    vp = (e_self * vp0 + jnp.sum(e_pair * vc, axis=1)) / denom
    return kp, vp


def _level1_body(kref, vref, k1ref, v1ref, *, Ts, Hc, D, scale):
    kc = kref[0].reshape(Ts // 2, 2, Hc, D)
    vc = vref[0].reshape(Ts // 2, 2, Hc, D)
    kp, vp = _parent_level(kc, vc, scale)
    k1ref[0] = kp
    v1ref[0] = vp


def _upper_body(k1ref, v1ref, k2ref, v2ref, *, S, Hc, D, levels, scale):
    ioff2 = _arr2_offsets(S, levels)
    for l in range(2, levels):
        n = S >> l
        if l == 2:
            kc = k1ref[0]
            vc = v1ref[0]
        else:
            kc = k2ref[0, ioff2[l - 1]:ioff2[l - 1] + 2 * n]
            vc = v2ref[0, ioff2[l - 1]:ioff2[l - 1] + 2 * n]
        kp, vp = _parent_level(kc.reshape(n, 2, Hc, D),
                               vc.reshape(n, 2, Hc, D), scale)
        k2ref[0, ioff2[l]:ioff2[l] + n] = kp
        v2ref[0, ioff2[l]:ioff2[l] + n] = vp


def _attn_body(*refs, T, H, D, levels, lt, scale, Npad):
    cols = levels + 1
    nlv = levels - 1
    qref, kref, vref, bref = refs[:4]
    knrefs = refs[4:4 + nlv]
    vnrefs = refs[4 + nlv:4 + 2 * nlv]
    oref, kscr, vscr = refs[4 + 2 * nlv:]

    bias = bref[...]

    tbase = [0, T]
    for l in range(2, lt):
        tbase.append(tbase[l - 1] + (T >> (l - 1)))
    sbase = tbase[lt - 1] + (T >> (lt - 1))
    nreal = sbase + (levels - lt)

    kscr[0:T] = kref[0]
    vscr[0:T] = vref[0]
    for l in range(1, levels):
        kl = knrefs[l - 1]
        vl = vnrefs[l - 1]
        if l < lt:
            kscr[tbase[l]:tbase[l] + (T >> l)] = kl[0]
            vscr[tbase[l]:tbase[l] + (T >> l)] = vl[0]
        else:
            p = sbase + (l - lt)
            kscr[p:p + 1] = kl[0]
            vscr[p:p + 1] = vl[0]
    if nreal < Npad:
        kscr[nreal:Npad] = jnp.zeros((Npad - nreal, H, D), jnp.float32)
        vscr[nreal:Npad] = jnp.zeros((Npad - nreal, H, D), jnp.float32)

    lane = lax.broadcasted_iota(jnp.int32, (T, Npad), 1)
    srow = lax.broadcasted_iota(jnp.int32, (T, 1), 0)
    neg = jnp.full((T, Npad), -jnp.inf, jnp.float32)
    mask = neg
    for c in range(cols):
        bias_c = bias[:, c:c + 1]
        if c == 0:
            pos = srow
        else:
            l = c - 1
            if l < lt:
                pos = tbase[l] + ((srow >> l) ^ 1)
            else:
                pos = jnp.full((T, 1), sbase + (l - lt), jnp.int32)
        mask = jnp.where(lane == pos, bias_c, mask)

    for h in range(H):
        qh = qref[0, :, h, :]
        kh = kscr[:, h, :]
        vh = vscr[:, h, :]
        s = lax.dot_general(qh, kh, (((1,), (1,)), ((), ())),
                            preferred_element_type=jnp.float32)
        s = s * scale + mask
        m = jnp.max(s, axis=1, keepdims=True)
        e = jnp.exp(s - m)
        den = jnp.sum(e, axis=1, keepdims=True)
        oh = lax.dot_general(e, vh, (((1,), (0,)), ((), ())),
                             preferred_element_type=jnp.float32)
        oref[0, :, h, :] = oh / den


def kernel(q, k, v, idx_map):
    B, S, H, D = q.shape
    cols = idx_map.shape[1]
    levels = cols - 1
    scale = 1.0 / math.sqrt(D)
    f32 = jnp.float32
    Hc = 8 if H % 8 == 0 else H
    ioff2 = _arr2_offsets(S, levels)
    par = pltpu.CompilerParams(dimension_semantics=("parallel",) * 3)
    par2 = pltpu.CompilerParams(dimension_semantics=("parallel", "parallel"))

    k = k.astype(f32)
    v = v.astype(f32)
    bias = jnp.where(idx_map < 0, -jnp.inf, 0.0).astype(f32)

    Ts = min(512, S)
    lvl1 = pl.pallas_call(
        functools.partial(_level1_body, Ts=Ts, Hc=Hc, D=D, scale=scale),
        grid=(B, S // Ts, H // Hc),
        in_specs=[
            pl.BlockSpec((1, Ts, Hc, D), lambda b, s, h: (b, s, h, 0)),
            pl.BlockSpec((1, Ts, Hc, D), lambda b, s, h: (b, s, h, 0)),
        ],
        out_specs=[
            pl.BlockSpec((1, Ts // 2, Hc, D), lambda b, s, h: (b, s, h, 0)),
            pl.BlockSpec((1, Ts // 2, Hc, D), lambda b, s, h: (b, s, h, 0)),
        ],
        out_shape=[
            jax.ShapeDtypeStruct((B, S // 2, H, D), f32),
            jax.ShapeDtypeStruct((B, S // 2, H, D), f32),
        ],
        compiler_params=par,
    )
    karr1, varr1 = lvl1(k, v)

    upper = pl.pallas_call(
        functools.partial(_upper_body, S=S, Hc=Hc, D=D, levels=levels,
                          scale=scale),
        grid=(B, H // Hc),
        in_specs=[
            pl.BlockSpec((1, S // 2, Hc, D), lambda b, h: (b, 0, h, 0)),
            pl.BlockSpec((1, S // 2, Hc, D), lambda b, h: (b, 0, h, 0)),
        ],
        out_specs=[
            pl.BlockSpec((1, S // 2, Hc, D), lambda b, h: (b, 0, h, 0)),
            pl.BlockSpec((1, S // 2, Hc, D), lambda b, h: (b, 0, h, 0)),
        ],
        out_shape=[
            jax.ShapeDtypeStruct((B, S // 2, H, D), f32),
            jax.ShapeDtypeStruct((B, S // 2, H, D), f32),
        ],
        compiler_params=par2,
    )
    karr2, varr2 = upper(karr1, varr1)

    T = min(128, S)
    lt = int(math.log2(T))
    nblocks = S // T
    nreal = 2 * T - 2 + (levels - lt)
    Npad = ((nreal + 127) // 128) * 128

    node_specs = []
    for l in range(1, levels):
        if l == 1:
            node_specs.append(pl.BlockSpec(
                (1, T >> 1, H, D), lambda b, t: (b, t, 0, 0)))
        elif l < lt:
            blk = T >> l
            base = ioff2[l] // blk
            node_specs.append(pl.BlockSpec(
                (1, blk, H, D),
                lambda b, t, base=base: (b, base + t, 0, 0)))
        else:
            node_specs.append(pl.BlockSpec(
                (1, 1, H, D),
                lambda b, t, l=l, off=ioff2[l]: (b, off + ((t >> (l - lt)) ^ 1),
                                                 0, 0)))

    attn = pl.pallas_call(
        functools.partial(_attn_body, T=T, H=H, D=D, levels=levels, lt=lt,
                          scale=scale, Npad=Npad),
        grid=(B, nblocks),
        in_specs=[
            pl.BlockSpec((1, T, H, D), lambda b, t: (b, t, 0, 0)),
            pl.BlockSpec((1, T, H, D), lambda b, t: (b, t, 0, 0)),
            pl.BlockSpec((1, T, H, D), lambda b, t: (b, t, 0, 0)),
            pl.BlockSpec((T, cols), lambda b, t: (t, 0)),
        ] + node_specs + node_specs,
        out_specs=pl.BlockSpec((1, T, H, D), lambda b, t: (b, t, 0, 0)),
        out_shape=jax.ShapeDtypeStruct((B, S, H, D), f32),
        scratch_shapes=[
            pltpu.VMEM((Npad, H, D), f32),
            pltpu.VMEM((Npad, H, D), f32),
        ],
        compiler_params=par2,
    )
    nreps = levels - 1
    karrs = [karr1 if l == 1 else karr2 for l in range(1, levels)]
    varrs = [varr1 if l == 1 else varr2 for l in range(1, levels)]
    out = attn(q.astype(f32), k, v, bias, *karrs, *varrs)
    return out.astype(q.dtype)


# speedup vs baseline: 5.2562x; 1.0235x over previous
import functools
import math

import jax
import jax.numpy as jnp
from jax import lax
from jax.experimental import pallas as pl
from jax.experimental.pallas import tpu as pltpu


def _arr2_offsets(S, levels):
    ioff2 = {2: 0}
    for l in range(3, levels):
        ioff2[l] = ioff2[l - 1] + (S >> (l - 1))
    return ioff2


def _parent_level(kc, vc, scale):
    kp0 = 0.5 * jnp.sum(kc, axis=1)
    vp0 = 0.5 * jnp.sum(vc, axis=1)
    s_self = jnp.sum(kp0 * kp0, axis=-1, keepdims=True) * scale
    s_pair = jnp.sum(kp0[:, None] * kc, axis=-1, keepdims=True) * scale
    m = jnp.maximum(s_self, jnp.max(s_pair, axis=1))
    e_self = jnp.exp(s_self - m)
    e_pair = jnp.exp(s_pair - m[:, None])
    denom = e_self + jnp.sum(e_pair, axis=1) + 1e-9
    kp = (e_self * kp0 + jnp.sum(e_pair * kc, axis=1)) / denom
    vp = (e_self * vp0 + jnp.sum(e_pair * vc, axis=1)) / denom
    return kp, vp


def _level1_body(kref, vref, k1ref, v1ref, *, Ts, Hc, D, scale):
    kc = kref[0].reshape(Ts // 2, 2, Hc, D)
    vc = vref[0].reshape(Ts // 2, 2, Hc, D)
    kp, vp = _parent_level(kc, vc, scale)
    k1ref[0] = kp
    v1ref[0] = vp


def _upper_body(k1ref, v1ref, k2ref, v2ref, *, S, Hc, D, levels, scale):
    ioff2 = _arr2_offsets(S, levels)
    for l in range(2, levels):
        n = S >> l
        if l == 2:
            kc = k1ref[0]
            vc = v1ref[0]
        else:
            kc = k2ref[0, ioff2[l - 1]:ioff2[l - 1] + 2 * n]
            vc = v2ref[0, ioff2[l - 1]:ioff2[l - 1] + 2 * n]
        kp, vp = _parent_level(kc.reshape(n, 2, Hc, D),
                               vc.reshape(n, 2, Hc, D), scale)
        k2ref[0, ioff2[l]:ioff2[l] + n] = kp
        v2ref[0, ioff2[l]:ioff2[l] + n] = vp


def _attn_body(qref, klref, vlref, bref, k1ref, v1ref, k2ref, v2ref, oref,
               kscr, vscr, *, T, H, D, S, levels, scale):
    cols = levels + 1
    lt = int(math.log2(T))
    ioff2 = _arr2_offsets(S, levels)
    t = pl.program_id(1)

    base1 = T
    base2 = T + (T >> 1)
    poff = {}
    off = base2
    for l in range(2, lt):
        poff[l] = off
        off += T >> l
    for l in range(lt, levels):
        poff[l] = off
        off += 1
    nused = off
    N = ((nused + 7) // 8) * 8

    kscr[:, 0:T] = klref[0]
    vscr[:, 0:T] = vlref[0]
    kscr[:, base1:base2] = k1ref[0]
    vscr[:, base1:base2] = v1ref[0]
    for l in range(2, levels):
        if l < lt:
            n = T >> l
            src = pl.ds(ioff2[l] + t * n, n)
        else:
            n = 1
            src = pl.ds(ioff2[l] + ((t >> (l - lt)) ^ 1), 1)
        kscr[:, poff[l]:poff[l] + n] = k2ref[0, :, src]
        vscr[:, poff[l]:poff[l] + n] = v2ref[0, :, src]
    if nused < N:
        z = jnp.zeros((H, N - nused, D), jnp.float32)
        kscr[:, nused:N] = z
        vscr[:, nused:N] = z

    bias = bref[...]
    lane = lax.broadcasted_iota(jnp.int32, (T, N), 1)
    srow = lax.broadcasted_iota(jnp.int32, (T, 1), 0)
    mask = jnp.full((T, N), -jnp.inf, jnp.float32)
    for c in range(cols):
        bias_c = bias[:, c:c + 1]
        if c == 0:
            pos = srow
        elif c == 1:
            pos = srow ^ 1
        elif c == 2:
            pos = base1 + ((srow >> 1) ^ 1)
        else:
            l = c - 1
            if l < lt:
                pos = poff[l] + ((srow >> l) ^ 1)
            else:
                pos = jnp.full((T, 1), poff[l], jnp.int32)
        mask = jnp.where(lane == pos, bias_c, mask)

    for h in range(H):
        qh = qref[0, :, h, :]
        kh = kscr[h]
        vh = vscr[h]
        s = lax.dot_general(qh, kh, (((1,), (1,)), ((), ())),
                            preferred_element_type=jnp.float32)
        s = s * scale + mask
        m = jnp.max(s, axis=1, keepdims=True)
        e = jnp.exp(s - m)
        den = jnp.sum(e, axis=1, keepdims=True)
        oh = lax.dot_general(e, vh, (((1,), (0,)), ((), ())),
                             preferred_element_type=jnp.float32)
        oref[0, :, h, :] = oh / den


def kernel(q, k, v, idx_map):
    B, S, H, D = q.shape
    cols = idx_map.shape[1]
    levels = cols - 1
    scale = 1.0 / math.sqrt(D)
    f32 = jnp.float32
    Hc = 8 if H % 8 == 0 else H
    T = min(128, S)
    nblocks = S // T
    par = pltpu.CompilerParams(dimension_semantics=("parallel",) * 3)
    par2 = pltpu.CompilerParams(dimension_semantics=("parallel", "parallel"))

    k = k.astype(f32)
    v = v.astype(f32)
    bias = jnp.where(idx_map < 0, -jnp.inf, 0.0).astype(f32)

    Ts = min(512, S)
    lvl1 = pl.pallas_call(
        functools.partial(_level1_body, Ts=Ts, Hc=Hc, D=D, scale=scale),
        grid=(B, S // Ts, H // Hc),
        in_specs=[
            pl.BlockSpec((1, Ts, Hc, D), lambda b, s, h: (b, s, h, 0)),
            pl.BlockSpec((1, Ts, Hc, D), lambda b, s, h: (b, s, h, 0)),
        ],
        out_specs=[
            pl.BlockSpec((1, Ts // 2, Hc, D), lambda b, s, h: (b, s, h, 0)),
            pl.BlockSpec((1, Ts // 2, Hc, D), lambda b, s, h: (b, s, h, 0)),
        ],
        out_shape=[
            jax.ShapeDtypeStruct((B, S // 2, H, D), f32),
            jax.ShapeDtypeStruct((B, S // 2, H, D), f32),
        ],
        compiler_params=par,
    )
    karr1, varr1 = lvl1(k, v)

    upper = pl.pallas_call(
        functools.partial(_upper_body, S=S, Hc=Hc, D=D, levels=levels,
                          scale=scale),
        grid=(B, H // Hc),
        in_specs=[
            pl.BlockSpec((1, S // 2, Hc, D), lambda b, h: (b, 0, h, 0)),
            pl.BlockSpec((1, S // 2, Hc, D), lambda b, h: (b, 0, h, 0)),
        ],
        out_specs=[
            pl.BlockSpec((1, S // 2, Hc, D), lambda b, h: (b, 0, h, 0)),
            pl.BlockSpec((1, S // 2, Hc, D), lambda b, h: (b, 0, h, 0)),
        ],
        out_shape=[
            jax.ShapeDtypeStruct((B, S // 2, H, D), f32),
            jax.ShapeDtypeStruct((B, S // 2, H, D), f32),
        ],
        compiler_params=par2,
    )
    karr2, varr2 = upper(karr1, varr1)

    kt = jnp.transpose(k, (0, 2, 1, 3))
    vt = jnp.transpose(v, (0, 2, 1, 3))
    k1t = jnp.transpose(karr1, (0, 2, 1, 3))
    v1t = jnp.transpose(varr1, (0, 2, 1, 3))
    k2t = jnp.transpose(karr2, (0, 2, 1, 3))
    v2t = jnp.transpose(varr2, (0, 2, 1, 3))

    nused = T + (T >> 1)
    lt = int(math.log2(T))
    for l in range(2, lt):
        nused += T >> l
    nused += levels - lt
    N = ((nused + 7) // 8) * 8

    attn = pl.pallas_call(
        functools.partial(_attn_body, T=T, H=H, D=D, S=S, levels=levels,
                          scale=scale),
        grid=(B, nblocks),
        in_specs=[
            pl.BlockSpec((1, T, H, D), lambda b, t: (b, t, 0, 0)),
            pl.BlockSpec((1, H, T, D), lambda b, t: (b, 0, t, 0)),
            pl.BlockSpec((1, H, T, D), lambda b, t: (b, 0, t, 0)),
            pl.BlockSpec((T, cols), lambda b, t: (t, 0)),
            pl.BlockSpec((1, H, T >> 1, D), lambda b, t: (b, 0, t, 0)),
            pl.BlockSpec((1, H, T >> 1, D), lambda b, t: (b, 0, t, 0)),
            pl.BlockSpec((1, H, S // 2, D), lambda b, t: (b, 0, 0, 0)),
            pl.BlockSpec((1, H, S // 2, D), lambda b, t: (b, 0, 0, 0)),
        ],
        out_specs=pl.BlockSpec((1, T, H, D), lambda b, t: (b, t, 0, 0)),
        out_shape=jax.ShapeDtypeStruct((B, S, H, D), f32),
        scratch_shapes=[
            pltpu.VMEM((H, N, D), f32),
            pltpu.VMEM((H, N, D), f32),
        ],
        compiler_params=par2,
    )
    out = attn(q.astype(f32), kt, vt, bias, k1t, v1t, k2t, v2t)
    return out.astype(q.dtype)
